# baseline (speedup 1.0000x reference)
import functools

import jax
import jax.numpy as jnp
from jax import lax
from jax.experimental import pallas as pl
from jax.experimental.pallas import tpu as pltpu
from jax.experimental.pallas import tpu_sc as plsc

_NUM_CORES = 2
_NUM_SUBCORES = 16
_NW = _NUM_CORES * _NUM_SUBCORES

_GW = 128
_CH = 512


@functools.partial(jax.jit, static_argnames=("n", "d"))
def _gather_rows(xf, table, n, d):
    per_w = n // _NW
    ng = per_w // _CH
    ks = _CH // _GW

    mesh = plsc.VectorSubcoreMesh(core_axis_name="c", subcore_axis_name="s")

    @functools.partial(
        pl.kernel,
        mesh=mesh,
        compiler_params=pltpu.CompilerParams(use_tc_tiling_on_sc=False),
        out_type=jax.ShapeDtypeStruct((n, d), jnp.float32),
        scratch_types=[
            pltpu.VMEM((per_w,), jnp.int32),
            pltpu.VMEM((_CH, d), jnp.float32),
            pltpu.VMEM((_CH, d), jnp.float32),
            pltpu.SemaphoreType.DMA,
            pltpu.SemaphoreType.DMA,
            pltpu.SemaphoreType.DMA,
        ],
    )
    def k(x_hbm, table_hbm, out_hbm, idx_v, rows0, rows1, gsem, osem0, osem1):
        wid = lax.axis_index("s") * _NUM_CORES + lax.axis_index("c")
        base = wid * per_w
        pltpu.sync_copy(x_hbm.at[pl.ds(base, per_w)], idx_v)

        bufs = ((rows0, osem0), (rows1, osem1))

        def do_chunk(g, rows, osem, wait_out):
            @pl.when(wait_out)
            def _():
                pltpu.make_async_copy(
                    rows, out_hbm.at[pl.ds(base, _CH)], osem
                ).wait()

            copies = []
            for j in range(ks):
                cp = pltpu.make_async_copy(
                    table_hbm.at[idx_v.at[pl.ds(g * _CH + j * _GW, _GW)]],
                    rows.at[pl.ds(j * _GW, _GW)],
                    gsem,
                )
                cp.start()
                copies.append(cp)
            for cp in copies:
                cp.wait()
            pltpu.make_async_copy(
                rows, out_hbm.at[pl.ds(base + g * _CH, _CH)], osem
            ).start()

        def pair_body(p, _):
            for b, (rows, osem) in enumerate(bufs):
                do_chunk(p * 2 + b, rows, osem, p > 0)
            return _

        lax.fori_loop(0, ng // 2, pair_body, None)

        for rows, osem in bufs:
            pltpu.make_async_copy(
                rows, out_hbm.at[pl.ds(base, _CH)], osem
            ).wait()

    return k(xf, table)


def kernel(x, table):
    b, h = x.shape
    v, d = table.shape
    n = b * h
    xf = x.reshape(n).astype(jnp.int32)
    out = _gather_rows(xf, table, n, d)
    return out.reshape(b, h, d)
